# baseline (device time: 22623 ns/iter reference)
import jax
import jax.numpy as jnp
from jax import lax
from jax.experimental import pallas as pl
from jax.experimental.pallas import tpu as pltpu

N_DEV = 32
G1 = 8
G2 = 4
BLK = 32


def kernel(x, w_mat):
    m, k_shard = x.shape
    k, n = w_mat.shape
    assert k_shard == BLK and m == N_DEV * BLK

    x4 = x.reshape(G2, G1, BLK, BLK)

    def body(
        x4_ref,
        w_hbm_ref,
        out_ref,
        stage1_ref,
        gather_ref,
        w_vmem_ref,
        send1,
        recv1,
        send2,
        recv2,
        w_sem,
    ):
        my = lax.axis_index("i")
        my1 = lax.div(my, G1)
        my2 = lax.rem(my, G1)

        barrier_sem = pltpu.get_barrier_semaphore()
        for st in range(1, G1):
            peer = my1 * G1 + lax.rem(my2 + st, G1)
            pl.semaphore_signal(
                barrier_sem,
                inc=1,
                device_id=(peer,),
                device_id_type=pl.DeviceIdType.MESH,
            )
        for st in range(1, G2):
            peer = lax.rem(my1 + st, G2) * G1 + my2
            pl.semaphore_signal(
                barrier_sem,
                inc=1,
                device_id=(peer,),
                device_id_type=pl.DeviceIdType.MESH,
            )

        w_copy = pltpu.make_async_copy(w_hbm_ref, w_vmem_ref, w_sem)
        w_copy.start()

        pl.semaphore_wait(barrier_sem, G1 - 1 + G2 - 1)

        sends1 = []
        for st in range(1, G1):
            d2 = lax.rem(my2 + st, G1)
            rdma = pltpu.make_async_remote_copy(
                src_ref=x4_ref.at[:, d2],
                dst_ref=stage1_ref.at[:, my2],
                send_sem=send1.at[st],
                recv_sem=recv1.at[st],
                device_id=(my1 * G1 + d2,),
                device_id_type=pl.DeviceIdType.MESH,
            )
            rdma.start()
            sends1.append(rdma)

        stage1_ref[:, my2] = x4_ref[:, my2]

        for st in range(1, G1):
            s2 = lax.rem(my2 + G1 - st, G1)
            pltpu.make_async_remote_copy(
                src_ref=x4_ref.at[:, 0],
                dst_ref=stage1_ref.at[:, s2],
                send_sem=send1.at[st],
                recv_sem=recv1.at[st],
                device_id=(my,),
                device_id_type=pl.DeviceIdType.MESH,
            ).wait_recv()

        sends2 = []
        for st in range(1, G2):
            d1 = lax.rem(my1 + st, G2)
            rdma = pltpu.make_async_remote_copy(
                src_ref=stage1_ref.at[d1],
                dst_ref=gather_ref.at[my1],
                send_sem=send2.at[st],
                recv_sem=recv2.at[st],
                device_id=(d1 * G1 + my2,),
                device_id_type=pl.DeviceIdType.MESH,
            )
            rdma.start()
            sends2.append(rdma)

        gather_ref[my1] = stage1_ref[my1]

        for st in range(1, G2):
            s1 = lax.rem(my1 + G2 - st, G2)
            pltpu.make_async_remote_copy(
                src_ref=stage1_ref.at[0],
                dst_ref=gather_ref.at[s1],
                send_sem=send2.at[st],
                recv_sem=recv2.at[st],
                device_id=(my,),
                device_id_type=pl.DeviceIdType.MESH,
            ).wait_recv()

        for rdma in sends1:
            rdma.wait_send()
        for rdma in sends2:
            rdma.wait_send()
        w_copy.wait()

        g = gather_ref[...]
        xrow = jnp.transpose(g, (2, 0, 1, 3)).reshape(BLK, k)
        y = jnp.dot(xrow, w_vmem_ref[...], preferred_element_type=jnp.float32)
        out_ref[...] = y * jax.nn.sigmoid(y)

    return pl.pallas_call(
        body,
        out_shape=jax.ShapeDtypeStruct((BLK, n), jnp.float32),
        in_specs=[
            pl.BlockSpec(memory_space=pltpu.VMEM),
            pl.BlockSpec(memory_space=pl.ANY),
        ],
        out_specs=pl.BlockSpec(memory_space=pltpu.VMEM),
        scratch_shapes=[
            pltpu.VMEM((G2, G1, BLK, BLK), jnp.float32),
            pltpu.VMEM((G2, G1, BLK, BLK), jnp.float32),
            pltpu.VMEM((1024, 1024), jnp.float32),
            pltpu.SemaphoreType.DMA((G1,)),
            pltpu.SemaphoreType.DMA((G1,)),
            pltpu.SemaphoreType.DMA((G2,)),
            pltpu.SemaphoreType.DMA((G2,)),
            pltpu.SemaphoreType.DMA,
        ],
        compiler_params=pltpu.CompilerParams(collective_id=0),
    )(x4, w_mat)
